# baseline (speedup 1.0000x reference)
import functools

import jax
import jax.numpy as jnp
from jax import lax
from jax.experimental import pallas as pl
from jax.experimental.pallas import tpu as pltpu
from jax.experimental.pallas import tpu_sc as plsc

N = 16384
B = 16
A = 100
S = 8
ND = 256
GD = 256
HD = 256
NEG = -1000000000.0

NC = 2
NS = 16
NW = NC * NS
PAIRS = B * A
PPT = PAIRS // NW
RPT = PPT * S
CHUNK = 100
NCHUNK = RPT // CHUNK


def _prep_body(batch_ref, sets_ref, p_ref, pt_ref, rows_ref, w_ref, valid_ref):
    bt = batch_ref[...]
    counts = [jnp.sum(jnp.where(bt == b, 1, 0)) for b in range(B)]
    offs = jnp.int32(0)
    for b in range(B):
        srow = sets_ref[b : b + 1, :]
        valid_ref[b : b + 1, :] = jnp.where(srow < counts[b], 1.0, 0.0)
        rows_ref[b : b + 1, :] = jnp.clip(srow + offs, 0, N - 1)
        offs = offs + counts[b]
    vf = valid_ref[...]
    cnt = lax.dot(vf, p_ref[...],
                  preferred_element_type=jnp.float32)
    inv = 1.0 / jnp.maximum(cnt, 1.0)
    w_ref[...] = vf * lax.dot(inv, pt_ref[...],
                              preferred_element_type=jnp.float32)


def _prep(batch2d, sets2d, pool, pool_t):
    return pl.pallas_call(
        _prep_body,
        out_shape=(
            jax.ShapeDtypeStruct((B, A * S), jnp.int32),
            jax.ShapeDtypeStruct((B, A * S), jnp.float32),
        ),
        scratch_shapes=[pltpu.VMEM((B, A * S), jnp.float32)],
    )(batch2d, sets2d, pool, pool_t)


def _sc_body(nf_hbm, idx_hbm, w_hbm, out_hbm, idx_v, w_v, rows_v, out_v, sem):
    wid = lax.axis_index("s") * NC + lax.axis_index("c")
    pltpu.sync_copy(idx_hbm.at[pl.ds(wid * NCHUNK, NCHUNK)], idx_v)
    pltpu.sync_copy(w_hbm.at[pl.ds(wid * RPT, RPT)], w_v.at[pl.ds(0, RPT)])
    descs = []
    for c in range(NCHUNK):
        descs.append(
            pltpu.async_copy(
                nf_hbm.at[idx_v.at[c]],
                rows_v.at[pl.ds(c * CHUNK, CHUNK)],
                sem,
            )
        )
    for d in descs:
        d.wait()

    def body(p, carry):
        base = p * S
        wvec = w_v[pl.ds(base, 16)]
        acc = [jnp.zeros((16,), jnp.float32) for _ in range(ND // 16)]
        for s in range(S):
            w = wvec[s]
            for d in range(ND // 16):
                acc[d] = acc[d] + w * rows_v[base + s, pl.ds(d * 16, 16)]
        for d in range(ND // 16):
            out_v[p, pl.ds(d * 16, 16)] = acc[d]
        return carry

    lax.fori_loop(0, PPT, body, jnp.int32(0))
    pltpu.sync_copy(out_v, out_hbm.at[pl.ds(wid * PPT, PPT)])


_sc_pool = pl.kernel(
    _sc_body,
    out_type=jax.ShapeDtypeStruct((PAIRS, ND), jnp.float32),
    mesh=plsc.VectorSubcoreMesh(core_axis_name="c", subcore_axis_name="s"),
    compiler_params=pltpu.CompilerParams(use_tc_tiling_on_sc=False),
    scratch_types=[
        pltpu.VMEM((NCHUNK, CHUNK), jnp.int32),
        pltpu.VMEM((RPT + 16,), jnp.float32),
        pltpu.VMEM((RPT, ND), jnp.float32),
        pltpu.VMEM((PPT, ND), jnp.float32),
        pltpu.SemaphoreType.DMA,
    ],
)


def _mlp_body(se_ref, g_ref, w1g_ref, w1n_ref, b1_ref, w2_ref, b2_ref,
              m_ref, out_ref):
    g = g_ref[...].reshape(1, GD)
    gw = lax.dot(g, w1g_ref[...], preferred_element_type=jnp.float32)
    se = se_ref[...].reshape(A, ND)
    h = lax.dot(se, w1n_ref[...], preferred_element_type=jnp.float32)
    h = jnp.maximum(h + gw + b1_ref[...], 0.0)
    val = jnp.sum(h * w2_ref[...], axis=1) + b2_ref[0, :]
    out_ref[0, 0, :] = jnp.where(m_ref[0, 0, :] > 0, val, NEG)


def _mlp(sep_emb3, gfeat3, w1g, w1n, b1row, w2row, b2row, maskf3):
    out = pl.pallas_call(
        _mlp_body,
        grid=(B,),
        in_specs=[
            pl.BlockSpec((1, A, ND), lambda b: (b, 0, 0)),
            pl.BlockSpec((1, 1, GD), lambda b: (b, 0, 0)),
            pl.BlockSpec((GD, HD), lambda b: (0, 0)),
            pl.BlockSpec((ND, HD), lambda b: (0, 0)),
            pl.BlockSpec((1, HD), lambda b: (0, 0)),
            pl.BlockSpec((1, HD), lambda b: (0, 0)),
            pl.BlockSpec((1, A), lambda b: (0, 0)),
            pl.BlockSpec((1, 1, A), lambda b: (b, 0, 0)),
        ],
        out_specs=pl.BlockSpec((1, 1, A), lambda b: (b, 0, 0)),
        out_shape=jax.ShapeDtypeStruct((B, 1, A), jnp.float32),
    )(sep_emb3, gfeat3, w1g, w1n, b1row, w2row, b2row, maskf3)
    return out.reshape(B, A)


def kernel(node_features, global_features, cube_mask, batch, sep_cube_sets,
           sep_mask, W1, b1, W2, b2):
    del cube_mask
    batch2d = batch.astype(jnp.int32).reshape(128, 128)
    sets2d = sep_cube_sets.astype(jnp.int32).reshape(B, A * S)
    pool = (jnp.arange(A * S, dtype=jnp.int32)[:, None] // S
            == jnp.arange(A, dtype=jnp.int32)[None, :]).astype(jnp.float32)
    pool_t = pool.T

    rows2d, w2d = _prep(batch2d, sets2d, pool, pool_t)
    idx_hbm = rows2d.reshape(NW * NCHUNK, CHUNK)
    w_flat = w2d.reshape(PAIRS * S)

    sep_emb = _sc_pool(node_features, idx_hbm, w_flat)

    w1g = W1[:GD, :]
    w1n = W1[GD:, :]
    b1row = b1[None, :]
    w2row = W2.reshape(1, HD)
    b2row = jnp.broadcast_to(b2.reshape(1, 1), (1, A))
    maskf3 = sep_mask.astype(jnp.float32).reshape(B, 1, A)

    return _mlp(sep_emb.reshape(B, A, ND), global_features.reshape(B, 1, GD),
                w1g, w1n, b1row, w2row, b2row, maskf3)


# speedup vs baseline: 1.0278x; 1.0278x over previous
import jax
import jax.numpy as jnp
from jax import lax
from jax.experimental import pallas as pl
from jax.experimental.pallas import tpu as pltpu
from jax.experimental.pallas import tpu_sc as plsc

N = 16384
B = 16
A = 100
S = 8
ND = 256
GD = 256
HD = 256
NEG = -1000000000.0

NC = 2
NS = 16
NW = NC * NS
PAIRS = B * A
PPT = PAIRS // NW
RPT = PPT * S
CROWS = 80
CPAIRS = CROWS // S
NCHUNK = RPT // CROWS
IR = PAIRS * S // CROWS
DV = ND // 16


def _prep_body(batch_ref, sets_ref, p_ref, pt_ref, rows_ref, w_ref):
    bt = batch_ref[...]
    counts = [jnp.sum(jnp.where(bt == b, 1, 0)) for b in range(B)]
    rowid = lax.broadcasted_iota(jnp.int32, (IR, 1), 0)
    cnt_col = jnp.zeros((IR, 1), jnp.int32)
    off_col = jnp.zeros((IR, 1), jnp.int32)
    offs = jnp.int32(0)
    rpb = IR // B
    for b in range(B):
        inb = (rowid >= b * rpb) & (rowid < (b + 1) * rpb)
        cnt_col = cnt_col + jnp.where(inb, counts[b], 0)
        off_col = off_col + jnp.where(inb, offs, 0)
        offs = offs + counts[b]
    sets = sets_ref[...]
    vf = jnp.where(sets < cnt_col, 1.0, 0.0)
    rows_ref[...] = jnp.clip(sets + off_col, 0, N - 1)
    cnt = lax.dot(vf, p_ref[...], preferred_element_type=jnp.float32)
    inv = 1.0 / jnp.maximum(cnt, 1.0)
    w_ref[...] = vf * lax.dot(inv, pt_ref[...],
                              preferred_element_type=jnp.float32)


def _prep(batch2d, sets2d, pool, pool_t):
    return pl.pallas_call(
        _prep_body,
        out_shape=(
            jax.ShapeDtypeStruct((IR, CROWS), jnp.int32),
            jax.ShapeDtypeStruct((IR, CROWS), jnp.float32),
        ),
    )(batch2d, sets2d, pool, pool_t)


def _sc_body(nf_hbm, idx_hbm, w_hbm, out_hbm, idx_v, w_v, rows_v, out_v,
             sem0, sem1):
    wid = lax.axis_index("s") * NC + lax.axis_index("c")
    pltpu.sync_copy(idx_hbm.at[pl.ds(wid * NCHUNK, NCHUNK)], idx_v)
    pltpu.sync_copy(w_hbm.at[pl.ds(wid * NCHUNK, NCHUNK)], w_v)
    sems = (sem0, sem1)
    descs = []

    def fire(c):
        buf = c % 2
        descs.append(
            pltpu.async_copy(nf_hbm.at[idx_v.at[c]], rows_v.at[buf],
                             sems[buf]))

    fire(0)
    for c in range(NCHUNK):
        if c + 1 < NCHUNK:
            fire(c + 1)
        descs[c].wait()
        buf = c % 2

        def body(jj, carry, buf=buf, c=c):
            wvec = w_v[c, pl.ds(jj * 16, 16)]
            for half in range(2):
                rbase = (jj * 2 + half) * S
                acc = [jnp.zeros((16,), jnp.float32) for _ in range(DV)]
                for s in range(S):
                    w = wvec[half * S + s]
                    for d in range(DV):
                        acc[d] = acc[d] + w * rows_v[buf, rbase + s,
                                                     pl.ds(d * 16, 16)]
                p = c * CPAIRS + jj * 2 + half
                for d in range(DV):
                    out_v[p, pl.ds(d * 16, 16)] = acc[d]
            return carry

        lax.fori_loop(0, CPAIRS // 2, body, jnp.int32(0))
    pltpu.sync_copy(out_v, out_hbm.at[pl.ds(wid * PPT, PPT)])


_sc_pool = pl.kernel(
    _sc_body,
    out_type=jax.ShapeDtypeStruct((PAIRS, ND), jnp.float32),
    mesh=plsc.VectorSubcoreMesh(core_axis_name="c", subcore_axis_name="s"),
    compiler_params=pltpu.CompilerParams(use_tc_tiling_on_sc=False),
    scratch_types=[
        pltpu.VMEM((NCHUNK, CROWS), jnp.int32),
        pltpu.VMEM((NCHUNK, CROWS), jnp.float32),
        pltpu.VMEM((2, CROWS, ND), jnp.float32),
        pltpu.VMEM((PPT, ND), jnp.float32),
        pltpu.SemaphoreType.DMA,
        pltpu.SemaphoreType.DMA,
    ],
)

MROW = 200
MG = PAIRS // MROW


def _mlp_body(se_ref, g_ref, w1g_ref, w1n_ref, b1_ref, w2_ref, b2_ref,
              m_ref, out_ref):
    se = se_ref[...]
    h = lax.dot(se, w1n_ref[...], preferred_element_type=jnp.float32)
    g2 = g_ref[...].reshape(2, GD)
    gw = lax.dot(g2, w1g_ref[...], preferred_element_type=jnp.float32)
    sel = lax.broadcasted_iota(jnp.int32, (MROW, 1), 0) < A
    gw_b = jnp.where(sel, gw[0:1, :], gw[1:2, :])
    h = jnp.maximum(h + gw_b + b1_ref[...], 0.0)
    val = jnp.sum(h * w2_ref[...], axis=1) + b2_ref[0, :]
    out_ref[0, 0, :] = jnp.where(m_ref[0, 0, :] > 0, val, NEG)


def _mlp(sep_emb, gfeat3, w1g, w1n, b1row, w2row, b2row, maskf3):
    out = pl.pallas_call(
        _mlp_body,
        grid=(MG,),
        in_specs=[
            pl.BlockSpec((MROW, ND), lambda g: (g, 0)),
            pl.BlockSpec((1, 2, GD), lambda g: (g, 0, 0)),
            pl.BlockSpec((GD, HD), lambda g: (0, 0)),
            pl.BlockSpec((ND, HD), lambda g: (0, 0)),
            pl.BlockSpec((1, HD), lambda g: (0, 0)),
            pl.BlockSpec((1, HD), lambda g: (0, 0)),
            pl.BlockSpec((1, MROW), lambda g: (0, 0)),
            pl.BlockSpec((1, 1, MROW), lambda g: (g, 0, 0)),
        ],
        out_specs=pl.BlockSpec((1, 1, MROW), lambda g: (g, 0, 0)),
        out_shape=jax.ShapeDtypeStruct((MG, 1, MROW), jnp.float32),
    )(sep_emb, gfeat3, w1g, w1n, b1row, w2row, b2row, maskf3)
    return out.reshape(B, A)


def kernel(node_features, global_features, cube_mask, batch, sep_cube_sets,
           sep_mask, W1, b1, W2, b2):
    del cube_mask
    batch2d = batch.astype(jnp.int32).reshape(128, 128)
    sets2d = sep_cube_sets.astype(jnp.int32).reshape(IR, CROWS)
    pool = (jnp.arange(CROWS, dtype=jnp.int32)[:, None] // S
            == jnp.arange(CPAIRS, dtype=jnp.int32)[None, :]
            ).astype(jnp.float32)
    pool_t = pool.T

    idx_hbm, w_hbm = _prep(batch2d, sets2d, pool, pool_t)

    sep_emb = _sc_pool(node_features, idx_hbm, w_hbm)

    w1g = W1[:GD, :]
    w1n = W1[GD:, :]
    b1row = b1[None, :]
    w2row = W2.reshape(1, HD)
    b2row = jnp.broadcast_to(b2.reshape(1, 1), (1, MROW))
    gfeat3 = global_features.reshape(MG, 2, GD)
    maskf3 = sep_mask.astype(jnp.float32).reshape(MG, 1, MROW)

    return _mlp(sep_emb, gfeat3, w1g, w1n, b1row, w2row, b2row, maskf3)
